# baseline (speedup 1.0000x reference)
import functools

import jax
import jax.numpy as jnp
from jax.experimental import pallas as pl
from jax.experimental.pallas import tpu as pltpu

N = 10000
BM = 200


def _pass1_kernel(a_ref, x_ref, w1_ref, b1_ref, w2_ref, g_ref, y1_ref):
    @pl.when(pl.program_id(0) == 0)
    def _():
        y1_ref[...] = jax.lax.dot_general(
            x_ref[...], w1_ref[...], (((1,), (0,)), ((), ())),
            preferred_element_type=jnp.float32,
            precision=jax.lax.Precision.HIGHEST,
        )

    z = jax.lax.dot_general(
        a_ref[...], y1_ref[...], (((1,), (0,)), ((), ())),
        preferred_element_type=jnp.float32,
    )
    h = jnp.maximum(z + b1_ref[...], 0.0)
    g_ref[...] = jax.lax.dot_general(
        h, w2_ref[...], (((1,), (0,)), ((), ())),
        preferred_element_type=jnp.float32,
        precision=jax.lax.Precision.HIGHEST,
    )


def _pass2_kernel(a_ref, g_ref, b2_ref, out_ref):
    out_ref[...] = jax.lax.dot_general(
        a_ref[...], g_ref[...], (((1,), (0,)), ((), ())),
        preferred_element_type=jnp.float32,
    ) + b2_ref[...]


@jax.jit
def kernel(x, norm_adj_mat, W1, b1, W2, b2):
    hid = W1.shape[1]
    ncls = W2.shape[1]
    grid = (N // BM,)

    g = pl.pallas_call(
        _pass1_kernel,
        grid=grid,
        in_specs=[
            pl.BlockSpec((BM, N), lambda i: (i, 0)),
            pl.BlockSpec((N, x.shape[1]), lambda i: (0, 0)),
            pl.BlockSpec((x.shape[1], hid), lambda i: (0, 0)),
            pl.BlockSpec((1, hid), lambda i: (0, 0)),
            pl.BlockSpec((hid, ncls), lambda i: (0, 0)),
        ],
        out_specs=pl.BlockSpec((BM, ncls), lambda i: (i, 0)),
        out_shape=jax.ShapeDtypeStruct((N, ncls), jnp.float32),
        scratch_shapes=[pltpu.VMEM((N, hid), jnp.float32)],
    )(norm_adj_mat, x, W1, b1.reshape(1, hid), W2)

    out = pl.pallas_call(
        _pass2_kernel,
        grid=grid,
        in_specs=[
            pl.BlockSpec((BM, N), lambda i: (i, 0)),
            pl.BlockSpec((N, ncls), lambda i: (0, 0)),
            pl.BlockSpec((1, ncls), lambda i: (0, 0)),
        ],
        out_specs=pl.BlockSpec((BM, ncls), lambda i: (i, 0)),
        out_shape=jax.ShapeDtypeStruct((N, ncls), jnp.float32),
    )(norm_adj_mat, g, b2.reshape(1, ncls))

    return out
